# Initial kernel scaffold; baseline (speedup 1.0000x reference)
import jax
import jax.numpy as jnp
from jax.experimental import pallas as pl


def kernel(feat, edge_index, W1, b1, W2, b2):
    raise NotImplementedError("write your pallas kernel here")


# speedup vs baseline: 8.4206x; 8.4206x over previous
import functools

import jax
import jax.numpy as jnp
from jax import lax
from jax.experimental import pallas as pl
from jax.experimental.pallas import tpu as pltpu
from jax.experimental.pallas import tpu_sc as plsc

N_NODES = 10000
N_EDGES = 320000
NFEAT = 128
NHID = 64

NPAD = 10240
EPAD = 327680
CHUNK = 128
NCH_TOT = EPAD // CHUNK
NCH_DEG = NCH_TOT // 32
NCH_AGG = NCH_TOT // 16
RPT = NPAD // 16


def _sc_mesh():
    return plsc.VectorSubcoreMesh(core_axis_name="c", subcore_axis_name="s")


def _sc_degrees(src2d, dst2d, zeros1d):
    @functools.partial(
        pl.kernel,
        out_type=(
            jax.ShapeDtypeStruct((2, NPAD), jnp.float32),
            jax.ShapeDtypeStruct((2, NPAD), jnp.float32),
        ),
        mesh=_sc_mesh(),
        compiler_params=pltpu.CompilerParams(use_tc_tiling_on_sc=False),
        scratch_types=[
            pltpu.VMEM((NCH_DEG, CHUNK), jnp.int32),
            pltpu.VMEM((NCH_DEG, CHUNK), jnp.int32),
            pltpu.VMEM((CHUNK,), jnp.float32),
            pltpu.VMEM_SHARED((NPAD,), jnp.float32),
            pltpu.VMEM_SHARED((NPAD,), jnp.float32),
        ],
    )
    def deg_kernel(src_hbm, dst_hbm, z_hbm, do_hbm, di_hbm,
                   src_v, dst_v, ones_v, acc_o, acc_i):
        c = lax.axis_index("c")
        s = lax.axis_index("s")
        wid = s * 2 + c
        pltpu.sync_copy(z_hbm.at[pl.ds(s * RPT, RPT)], acc_o.at[pl.ds(s * RPT, RPT)])
        pltpu.sync_copy(z_hbm.at[pl.ds(s * RPT, RPT)], acc_i.at[pl.ds(s * RPT, RPT)])
        pltpu.sync_copy(src_hbm.at[pl.ds(wid * NCH_DEG, NCH_DEG)], src_v)
        pltpu.sync_copy(dst_hbm.at[pl.ds(wid * NCH_DEG, NCH_DEG)], dst_v)
        for k in range(CHUNK // 16):
            ones_v[pl.ds(k * 16, 16)] = jnp.ones((16,), jnp.float32)
        plsc.subcore_barrier()

        def body(j, carry):
            pltpu.sync_copy(ones_v, acc_o.at[src_v.at[j]], add=True)
            pltpu.sync_copy(ones_v, acc_i.at[dst_v.at[j]], add=True)
            return carry

        lax.fori_loop(0, NCH_DEG, body, 0)
        plsc.subcore_barrier()
        pltpu.sync_copy(acc_o.at[pl.ds(s * RPT, RPT)], do_hbm.at[c, pl.ds(s * RPT, RPT)])
        pltpu.sync_copy(acc_i.at[pl.ds(s * RPT, RPT)], di_hbm.at[c, pl.ds(s * RPT, RPT)])

    return deg_kernel(src2d, dst2d, zeros1d)


def _sc_aggregate(hcat, srcoff, dst2d, zeros2d, D):
    @functools.partial(
        pl.kernel,
        out_type=jax.ShapeDtypeStruct((2, NPAD, D), jnp.float32),
        mesh=_sc_mesh(),
        compiler_params=pltpu.CompilerParams(use_tc_tiling_on_sc=False),
        scratch_types=[
            pltpu.VMEM((NCH_AGG, CHUNK), jnp.int32),
            pltpu.VMEM((NCH_AGG, CHUNK), jnp.int32),
            pltpu.VMEM((2, CHUNK, D), jnp.float32),
            pltpu.VMEM_SHARED((NPAD, D), jnp.float32),
            pltpu.SemaphoreType.DMA,
            pltpu.SemaphoreType.DMA,
        ],
    )
    def agg_kernel(h_hbm, src_hbm, dst_hbm, z_hbm, out_hbm,
                   src_v, dst_v, rows_v, acc, sem0, sem1):
        c = lax.axis_index("c")
        s = lax.axis_index("s")
        pltpu.sync_copy(z_hbm.at[pl.ds(s * RPT, RPT)], acc.at[pl.ds(s * RPT, RPT)])
        pltpu.sync_copy(src_hbm.at[c, pl.ds(s * NCH_AGG, NCH_AGG)], src_v)
        pltpu.sync_copy(dst_hbm.at[pl.ds(s * NCH_AGG, NCH_AGG)], dst_v)
        plsc.subcore_barrier()

        pltpu.async_copy(h_hbm.at[src_v.at[0]], rows_v.at[0], sem0)

        def body(j, carry):
            @pl.when(j < NCH_AGG - 1)
            def _prefetch():
                @pl.when(j % 2 == 0)
                def _():
                    pltpu.async_copy(h_hbm.at[src_v.at[j + 1]], rows_v.at[1], sem1)

                @pl.when(j % 2 == 1)
                def _():
                    pltpu.async_copy(h_hbm.at[src_v.at[j + 1]], rows_v.at[0], sem0)

            @pl.when(j % 2 == 0)
            def _even():
                pltpu.make_async_copy(h_hbm.at[src_v.at[j]], rows_v.at[0], sem0).wait()
                pltpu.sync_copy(rows_v.at[0], acc.at[dst_v.at[j]], add=True)

            @pl.when(j % 2 == 1)
            def _odd():
                pltpu.make_async_copy(h_hbm.at[src_v.at[j]], rows_v.at[1], sem1).wait()
                pltpu.sync_copy(rows_v.at[1], acc.at[dst_v.at[j]], add=True)

            return carry

        lax.fori_loop(0, NCH_AGG, body, 0)
        plsc.subcore_barrier()
        pltpu.sync_copy(acc.at[pl.ds(s * RPT, RPT)], out_hbm.at[c, pl.ds(s * RPT, RPT)])

    return agg_kernel(hcat, srcoff, dst2d, zeros2d)


_BLK = 1024


def _tc_prep(feat_pad, degs_t):
    def body(f_ref, d_ref, h1_ref, do_ref, di_ref):
        d = d_ref[...]
        d_out = lax.rsqrt(jnp.clip(d[:, 0:1] + d[:, 1:2], 1.0, None))
        d_in = lax.rsqrt(jnp.clip(d[:, 2:3] + d[:, 3:4], 1.0, None))
        h = f_ref[...] * d_out
        h1_ref[0] = h[:, :NFEAT // 2]
        h1_ref[1] = h[:, NFEAT // 2:]
        do_ref[...] = d_out
        di_ref[...] = d_in

    grid = NPAD // _BLK
    return pl.pallas_call(
        body,
        grid=(grid,),
        in_specs=[
            pl.BlockSpec((_BLK, NFEAT), lambda i: (i, 0)),
            pl.BlockSpec((_BLK, 4), lambda i: (i, 0)),
        ],
        out_specs=[
            pl.BlockSpec((2, _BLK, NFEAT // 2), lambda i: (0, i, 0)),
            pl.BlockSpec((_BLK, 1), lambda i: (i, 0)),
            pl.BlockSpec((_BLK, 1), lambda i: (i, 0)),
        ],
        out_shape=[
            jax.ShapeDtypeStruct((2, NPAD, NFEAT // 2), jnp.float32),
            jax.ShapeDtypeStruct((NPAD, 1), jnp.float32),
            jax.ShapeDtypeStruct((NPAD, 1), jnp.float32),
        ],
    )(feat_pad, degs_t)


def _tc_mid(p, d_in, d_out, W1, b1, W2):
    def body(p0_ref, p1_ref, di_ref, do_ref, w1_ref, b1_ref, w2_ref, h2_ref):
        agg = jnp.concatenate([p0_ref[0], p1_ref[0]], axis=1) * di_ref[...]
        x = jnp.maximum(
            jnp.dot(agg, w1_ref[...], preferred_element_type=jnp.float32)
            + b1_ref[...], 0.0)
        h2 = jnp.dot(x * do_ref[...], w2_ref[...],
                     preferred_element_type=jnp.float32)
        h2_ref[0] = h2[:, :NHID // 2]
        h2_ref[1] = h2[:, NHID // 2:]

    grid = NPAD // _BLK
    return pl.pallas_call(
        body,
        grid=(grid,),
        in_specs=[
            pl.BlockSpec((1, _BLK, NFEAT // 2), lambda i: (0, i, 0)),
            pl.BlockSpec((1, _BLK, NFEAT // 2), lambda i: (1, i, 0)),
            pl.BlockSpec((_BLK, 1), lambda i: (i, 0)),
            pl.BlockSpec((_BLK, 1), lambda i: (i, 0)),
            pl.BlockSpec((NFEAT, 2 * NHID), lambda i: (0, 0)),
            pl.BlockSpec((1, 2 * NHID), lambda i: (0, 0)),
            pl.BlockSpec((2 * NHID, NHID), lambda i: (0, 0)),
        ],
        out_specs=pl.BlockSpec((2, _BLK, NHID // 2), lambda i: (0, i, 0)),
        out_shape=jax.ShapeDtypeStruct((2, NPAD, NHID // 2), jnp.float32),
    )(p, p, d_in, d_out, W1, b1, W2)


def _tc_final(q, d_in, b2):
    def body(q0_ref, q1_ref, di_ref, b2_ref, o_ref):
        z = (jnp.concatenate([q0_ref[0], q1_ref[0]], axis=1) * di_ref[...]
             + b2_ref[...])
        m = jnp.max(z, axis=1, keepdims=True)
        e = jnp.exp(z - m)
        lse = jnp.log(jnp.sum(e, axis=1, keepdims=True))
        o_ref[...] = z - m - lse

    grid = NPAD // _BLK
    return pl.pallas_call(
        body,
        grid=(grid,),
        in_specs=[
            pl.BlockSpec((1, _BLK, NHID // 2), lambda i: (0, i, 0)),
            pl.BlockSpec((1, _BLK, NHID // 2), lambda i: (1, i, 0)),
            pl.BlockSpec((_BLK, 1), lambda i: (i, 0)),
            pl.BlockSpec((1, NHID), lambda i: (0, 0)),
        ],
        out_specs=pl.BlockSpec((_BLK, NHID), lambda i: (i, 0)),
        out_shape=jax.ShapeDtypeStruct((NPAD, NHID), jnp.float32),
    )(q, q, d_in, b2)


def kernel(feat, edge_index, W1, b1, W2, b2):
    n_junk = NPAD - N_NODES
    n_pad_e = EPAD - N_EDGES
    i = jnp.arange(n_pad_e, dtype=jnp.int32)
    pad_src = N_NODES + (i % n_junk)
    pad_dst = N_NODES + ((i + n_junk // 2) % n_junk)
    src = jnp.concatenate([edge_index[0].astype(jnp.int32), pad_src])
    dst2d = jnp.concatenate([edge_index[1].astype(jnp.int32), pad_dst]).reshape(
        NCH_TOT, CHUNK)
    src2d = src.reshape(NCH_TOT, CHUNK)
    srcoff = jnp.stack([src2d, src2d + NPAD])

    feat_pad = jnp.zeros((NPAD, NFEAT), jnp.float32).at[:N_NODES].set(feat)
    zeros1d = jnp.zeros((NPAD,), jnp.float32)
    zeros64 = jnp.zeros((NPAD, NFEAT // 2), jnp.float32)
    zeros32 = jnp.zeros((NPAD, NHID // 2), jnp.float32)

    deg_o, deg_i = _sc_degrees(src2d, dst2d, zeros1d)
    degs_t = jnp.concatenate([deg_o, deg_i], axis=0).T

    h1cat, d_out, d_in = _tc_prep(feat_pad, degs_t)
    p = _sc_aggregate(h1cat.reshape(2 * NPAD, NFEAT // 2), srcoff, dst2d,
                      zeros64, NFEAT // 2)
    h2cat = _tc_mid(p, d_in, d_out, W1, b1.reshape(1, -1), W2)
    q = _sc_aggregate(h2cat.reshape(2 * NPAD, NHID // 2), srcoff, dst2d,
                      zeros32, NHID // 2)
    out = _tc_final(q, d_in, b2.reshape(1, -1))
    return out[:N_NODES]
